# initial kernel scaffold (unmeasured)
import jax
import jax.numpy as jnp
from jax import lax
from jax.experimental import pallas as pl
from jax.experimental.pallas import tpu as pltpu


def kernel(
    x,
):
    def body(*refs):
        pass

    out_shape = jax.ShapeDtypeStruct(..., jnp.float32)
    return pl.pallas_call(body, out_shape=out_shape)(...)


# baseline (device time: 11892 ns/iter reference)
import functools

import jax
import jax.numpy as jnp
from jax import lax
from jax.experimental import pallas as pl
from jax.experimental.pallas import tpu as pltpu

M = 512


def kernel(x):
    m_per, n = x.shape
    assert m_per == M and n == 2 * M

    def body(x_ref, out_ref, send_buf, send_sem, recv_sem):
        my_x = lax.axis_index("x")
        my_y = lax.axis_index("y")

        barrier_sem = pltpu.get_barrier_semaphore()
        pl.semaphore_signal(
            barrier_sem, inc=1,
            device_id=(1 - my_x, my_y),
            device_id_type=pl.DeviceIdType.MESH,
        )
        pl.semaphore_wait(barrier_sem, 1)

        def exchange(px):
            pp = 1 - px
            send_buf[...] = x_ref[:, pp * M:(pp + 1) * M].astype(jnp.bfloat16)
            rdma = pltpu.make_async_remote_copy(
                src_ref=send_buf,
                dst_ref=out_ref.at[pl.ds(px * M, M), :],
                send_sem=send_sem,
                recv_sem=recv_sem,
                device_id=(pp, my_y),
                device_id_type=pl.DeviceIdType.MESH,
            )
            rdma.start()
            out_ref[pl.ds(px * M, M), :] = (
                x_ref[:, px * M:(px + 1) * M].astype(jnp.bfloat16)
            )
            rdma.wait()

        pl.when(my_x == 0)(functools.partial(exchange, 0))
        pl.when(my_x == 1)(functools.partial(exchange, 1))

    return pl.pallas_call(
        body,
        out_shape=jax.ShapeDtypeStruct((2 * M, M), jnp.bfloat16),
        in_specs=[pl.BlockSpec(memory_space=pltpu.VMEM)],
        out_specs=pl.BlockSpec(memory_space=pltpu.VMEM),
        scratch_shapes=[
            pltpu.VMEM((M, M), jnp.bfloat16),
            pltpu.SemaphoreType.DMA,
            pltpu.SemaphoreType.DMA,
        ],
        compiler_params=pltpu.CompilerParams(collective_id=0),
    )(x)


# device time: 11846 ns/iter; 1.0039x vs baseline; 1.0039x over previous
import functools

import jax
import jax.numpy as jnp
from jax import lax
from jax.experimental import pallas as pl
from jax.experimental.pallas import tpu as pltpu

M = 512
N_CHUNKS = 4
CHUNK = M // N_CHUNKS


def kernel(x):
    m_per, n = x.shape
    assert m_per == M and n == 2 * M

    def body(x_ref, out_ref, send_buf, send_sem, recv_sem):
        my_x = lax.axis_index("x")
        my_y = lax.axis_index("y")

        barrier_sem = pltpu.get_barrier_semaphore()
        pl.semaphore_signal(
            barrier_sem, inc=1,
            device_id=(1 - my_x, my_y),
            device_id_type=pl.DeviceIdType.MESH,
        )
        pl.semaphore_wait(barrier_sem, 1)

        def exchange(px):
            pp = 1 - px

            def chunk_rdma(k):
                r = k * CHUNK
                return pltpu.make_async_remote_copy(
                    src_ref=send_buf.at[pl.ds(r, CHUNK), :],
                    dst_ref=out_ref.at[pl.ds(px * M + r, CHUNK), :],
                    send_sem=send_sem.at[k],
                    recv_sem=recv_sem.at[k],
                    device_id=(pp, my_y),
                    device_id_type=pl.DeviceIdType.MESH,
                )

            for k in range(N_CHUNKS):
                r = k * CHUNK
                send_buf[pl.ds(r, CHUNK), :] = (
                    x_ref[r:r + CHUNK, pp * M:(pp + 1) * M].astype(jnp.bfloat16)
                )
                chunk_rdma(k).start()
            out_ref[pl.ds(px * M, M), :] = (
                x_ref[:, px * M:(px + 1) * M].astype(jnp.bfloat16)
            )
            for k in range(N_CHUNKS):
                chunk_rdma(k).wait()

        pl.when(my_x == 0)(functools.partial(exchange, 0))
        pl.when(my_x == 1)(functools.partial(exchange, 1))

    return pl.pallas_call(
        body,
        out_shape=jax.ShapeDtypeStruct((2 * M, M), jnp.bfloat16),
        in_specs=[pl.BlockSpec(memory_space=pltpu.VMEM)],
        out_specs=pl.BlockSpec(memory_space=pltpu.VMEM),
        scratch_shapes=[
            pltpu.VMEM((M, M), jnp.bfloat16),
            pltpu.SemaphoreType.DMA((N_CHUNKS,)),
            pltpu.SemaphoreType.DMA((N_CHUNKS,)),
        ],
        compiler_params=pltpu.CompilerParams(collective_id=0),
    )(x)
